# baseline (speedup 1.0000x reference)
import functools

import jax
import jax.numpy as jnp
from jax import lax
from jax.experimental import pallas as pl
from jax.experimental.pallas import tpu as pltpu
from jax.experimental.pallas import tpu_sc as plsc

VOCAB = 1000000
EMBED = 64
B, L = 4096, 200
N = B * L
NC = 2
NS = 16
NW = NC * NS
BPW = N // NW
CH = 128
NCH = BPW // CH

_mesh = plsc.VectorSubcoreMesh(core_axis_name="c", subcore_axis_name="s")
_params = pltpu.CompilerParams(
    use_tc_tiling_on_sc=True, needs_layout_passes=False)


@functools.partial(
    pl.kernel,
    mesh=_mesh,
    out_type=jax.ShapeDtypeStruct((N, EMBED), jnp.float32),
    scratch_types=[
        pltpu.VMEM((NCH, CH), jnp.int32),
        pltpu.VMEM((CH,), jnp.int32),
        pltpu.VMEM((CH,), jnp.int32),
        pltpu.VMEM((CH, 1, 2 * EMBED), jnp.float32),
        pltpu.VMEM((CH, 1, 2 * EMBED), jnp.float32),
        pltpu.VMEM((CH, EMBED), jnp.float32),
        pltpu.SemaphoreType.DMA,
        pltpu.SemaphoreType.DMA,
        pltpu.SemaphoreType.DMA,
    ],
    compiler_params=_params,
)
def _gather_rows(idx_hbm, t3_hbm, out_hbm, idx_v, h0, h1, p0, p1, rows_v,
                 gsem0, gsem1, wsem):
    wid = lax.axis_index("s") * NC + lax.axis_index("c")
    base = wid * BPW
    hs = (h0, h1)
    ps = (p0, p1)
    gsems = (gsem0, gsem1)
    pltpu.sync_copy(idx_hbm.at[wid], idx_v)

    def _start_gather(j, buf_i):
        for cc in range(8):
            raw = idx_v[j, pl.ds(cc * 16, 16)]
            hs[buf_i][pl.ds(cc * 16, 16)] = jax.lax.shift_right_logical(raw, 1)
        pltpu.async_copy(t3_hbm.at[hs[buf_i]], ps[buf_i], gsems[buf_i])

    _start_gather(0, 0)

    def body(j2, carry):
        for t in range(2):
            j = j2 * 2 + t

            @pl.when(j + 1 < NCH)
            def _():
                _start_gather(j + 1, 1 - t)

            pltpu.make_async_copy(t3_hbm.at[hs[t]], ps[t], gsems[t]).wait()
            slabs = ps[t]

            @pl.when(j >= 1)
            def _():
                pltpu.make_async_copy(
                    rows_v, out_hbm.at[pl.ds(0, CH)], wsem).wait()

            @plsc.parallel_loop(0, CH // 16, unroll=2)
            def sel(g):
                hv = jax.lax.shift_left(
                    jax.lax.bitwise_and(idx_v[j, pl.ds(g * 16, 16)], 1), 6)
                for i in range(16):
                    c = g * 16 + i
                    h64 = hv[i]
                    for m in range(4):
                        rows_v[c, pl.ds(m * 16, 16)] = (
                            slabs[c, 0, pl.ds(h64 + m * 16, 16)])

            pltpu.async_copy(
                rows_v, out_hbm.at[pl.ds(base + j * CH, CH)], wsem)

        return carry

    lax.fori_loop(0, NCH // 2, body, 0)
    pltpu.make_async_copy(rows_v, out_hbm.at[pl.ds(0, CH)], wsem).wait()


def kernel(x, table):
    t3 = table.reshape(VOCAB // 2, 1, 2 * EMBED)
    idx3 = x.astype(jnp.int32).reshape(NW, NCH, CH)
    out = _gather_rows(idx3, t3)
    return out.reshape(B, L, EMBED)
